# baseline (speedup 1.0000x reference)
import jax
import jax.numpy as jnp
from jax import lax
from jax.experimental import pallas as pl
from jax.experimental.pallas import tpu as pltpu
from jax.experimental.pallas import tpu_sc as plsc

BATCH = 16384
EMBED_DIM = 64
NUM_CORES = 2
NUM_SUBCORES = 16
NUM_WORKERS = NUM_CORES * NUM_SUBCORES
B_PER_W = BATCH // NUM_WORKERS
CHUNK = 128
CHUNKS_PER_W = B_PER_W // CHUNK
LANES = 16
VECS_PER_ROW = EMBED_DIM // LANES


def _body(node_hbm, rel_hbm, head_hbm, relids_hbm, tail_hbm, out_hbm,
          idx_h, idx_r, idx_t, h_buf, r_buf, t_buf, sem):
    wid = lax.axis_index("s") * NUM_CORES + lax.axis_index("c")
    idx_row = wid * CHUNKS_PER_W

    pltpu.sync_copy(head_hbm.at[pl.ds(idx_row, CHUNKS_PER_W)], idx_h)
    pltpu.sync_copy(relids_hbm.at[pl.ds(idx_row, CHUNKS_PER_W)], idx_r)
    pltpu.sync_copy(tail_hbm.at[pl.ds(idx_row, CHUNKS_PER_W)], idx_t)

    copies = []
    for j in range(CHUNKS_PER_W):
        dst = pl.ds(j * CHUNK, CHUNK)
        copies.append(pltpu.async_copy(node_hbm.at[idx_h.at[j]],
                                       h_buf.at[dst], sem))
        copies.append(pltpu.async_copy(rel_hbm.at[idx_r.at[j]],
                                       r_buf.at[dst], sem))
        copies.append(pltpu.async_copy(node_hbm.at[idx_t.at[j]],
                                       t_buf.at[dst], sem))
    for c in copies:
        c.wait()

    def row_body(i, carry):
        for j in range(VECS_PER_ROW):
            sl = pl.ds(j * LANES, LANES)
            h_buf[i, sl] = h_buf[i, sl] + r_buf[i, sl] + t_buf[i, sl]
        return carry

    lax.fori_loop(0, B_PER_W, row_body, 0)

    pltpu.sync_copy(h_buf, out_hbm.at[pl.ds(wid * B_PER_W, B_PER_W)])


@jax.jit
def kernel(head_ids, rel_ids, tail_ids, node_embeddings, rel_weight):
    mesh = plsc.VectorSubcoreMesh(core_axis_name="c", subcore_axis_name="s",
                                  num_cores=NUM_CORES,
                                  num_subcores=NUM_SUBCORES)
    k = pl.kernel(
        _body,
        out_type=jax.ShapeDtypeStruct((BATCH, EMBED_DIM), jnp.float32),
        mesh=mesh,
        compiler_params=pltpu.CompilerParams(use_tc_tiling_on_sc=False),
        scratch_types=[
            pltpu.VMEM((CHUNKS_PER_W, CHUNK), jnp.int32),
            pltpu.VMEM((CHUNKS_PER_W, CHUNK), jnp.int32),
            pltpu.VMEM((CHUNKS_PER_W, CHUNK), jnp.int32),
            pltpu.VMEM((B_PER_W, EMBED_DIM), jnp.float32),
            pltpu.VMEM((B_PER_W, EMBED_DIM), jnp.float32),
            pltpu.VMEM((B_PER_W, EMBED_DIM), jnp.float32),
            pltpu.SemaphoreType.DMA,
        ],
    )
    head2d = head_ids.reshape(NUM_WORKERS * CHUNKS_PER_W, CHUNK)
    rel2d = rel_ids.reshape(NUM_WORKERS * CHUNKS_PER_W, CHUNK)
    tail2d = tail_ids.reshape(NUM_WORKERS * CHUNKS_PER_W, CHUNK)
    return k(node_embeddings, rel_weight, head2d, rel2d, tail2d)


# speedup vs baseline: 1.1108x; 1.1108x over previous
import jax
import jax.numpy as jnp
from jax import lax
from jax.experimental import pallas as pl
from jax.experimental.pallas import tpu as pltpu
from jax.experimental.pallas import tpu_sc as plsc

BATCH = 16384
EMBED_DIM = 64
EMBED_PAD = 128
NUM_CORES = 2
NUM_SUBCORES = 16
NUM_WORKERS = NUM_CORES * NUM_SUBCORES
B_PER_W = BATCH // NUM_WORKERS
HALF = B_PER_W // 2
CHUNK = 128
CHUNKS_PER_HALF = HALF // CHUNK
LANES = 16
VECS_PER_ROW = EMBED_DIM // LANES


def _body(node_hbm, rel_hbm, head_hbm, relids_hbm, tail_hbm, out_hbm,
          idx_h, idx_r, idx_t, h_buf, r_buf, t_buf, sem):
    wid = lax.axis_index("s") * NUM_CORES + lax.axis_index("c")

    for half in range(2):
        idx_row = wid * 2 * CHUNKS_PER_HALF + half * CHUNKS_PER_HALF
        base = wid * B_PER_W + half * HALF

        pltpu.sync_copy(head_hbm.at[pl.ds(idx_row, CHUNKS_PER_HALF)], idx_h)
        pltpu.sync_copy(relids_hbm.at[pl.ds(idx_row, CHUNKS_PER_HALF)], idx_r)
        pltpu.sync_copy(tail_hbm.at[pl.ds(idx_row, CHUNKS_PER_HALF)], idx_t)

        copies = []
        for j in range(CHUNKS_PER_HALF):
            dst = pl.ds(j * CHUNK, CHUNK)
            copies.append(pltpu.async_copy(node_hbm.at[idx_h.at[j]],
                                           h_buf.at[dst], sem))
            copies.append(pltpu.async_copy(rel_hbm.at[idx_r.at[j]],
                                           r_buf.at[dst], sem))
            copies.append(pltpu.async_copy(node_hbm.at[idx_t.at[j]],
                                           t_buf.at[dst], sem))
        for c in copies:
            c.wait()

        def row_body(i, carry):
            for j in range(VECS_PER_ROW):
                sl = pl.ds(j * LANES, LANES)
                h_buf[i, sl] = h_buf[i, sl] + r_buf[i, sl] + t_buf[i, sl]
            return carry

        lax.fori_loop(0, HALF, row_body, 0)

        pltpu.sync_copy(h_buf, out_hbm.at[pl.ds(base, HALF)])


@jax.jit
def kernel(head_ids, rel_ids, tail_ids, node_embeddings, rel_weight):
    mesh = plsc.VectorSubcoreMesh(core_axis_name="c", subcore_axis_name="s",
                                  num_cores=NUM_CORES,
                                  num_subcores=NUM_SUBCORES)
    k = pl.kernel(
        _body,
        out_type=jax.ShapeDtypeStruct((BATCH, EMBED_PAD), jnp.float32),
        mesh=mesh,
        scratch_types=[
            pltpu.VMEM((CHUNKS_PER_HALF, CHUNK), jnp.int32),
            pltpu.VMEM((CHUNKS_PER_HALF, CHUNK), jnp.int32),
            pltpu.VMEM((CHUNKS_PER_HALF, CHUNK), jnp.int32),
            pltpu.VMEM((HALF, EMBED_PAD), jnp.float32),
            pltpu.VMEM((HALF, EMBED_PAD), jnp.float32),
            pltpu.VMEM((HALF, EMBED_PAD), jnp.float32),
            pltpu.SemaphoreType.DMA,
        ],
    )
    node_pad = jnp.pad(node_embeddings, ((0, 0), (0, EMBED_PAD - EMBED_DIM)))
    rel_pad = jnp.pad(rel_weight, ((0, 0), (0, EMBED_PAD - EMBED_DIM)))
    nrows = NUM_WORKERS * 2 * CHUNKS_PER_HALF
    head2d = head_ids.reshape(nrows, CHUNK)
    rel2d = rel_ids.reshape(nrows, CHUNK)
    tail2d = tail_ids.reshape(nrows, CHUNK)
    out = k(node_pad, rel_pad, head2d, rel2d, tail2d)
    return out[:, :EMBED_DIM]
